# baseline (speedup 1.0000x reference)
import functools

import jax
import jax.numpy as jnp
from jax import lax
from jax.experimental import pallas as pl
from jax.experimental.pallas import tpu as pltpu

F32 = jnp.float32


def _supports_kernel(e_ref, s1_ref, s2_ref):
    E = e_ref[...]
    A = lax.dot_general(E, E, (((1,), (1,)), ((), ())),
                        preferred_element_type=F32)
    A = jnp.maximum(A, 0.0)
    A = A - jnp.max(A, axis=1, keepdims=True)
    eA = jnp.exp(A)
    S = eA / jnp.sum(eA, axis=1, keepdims=True)
    n = S.shape[0]
    row = lax.broadcasted_iota(jnp.int32, (n, n), 0)
    col = lax.broadcasted_iota(jnp.int32, (n, n), 1)
    eye = (row == col).astype(F32)
    s1_ref[...] = S
    s2_ref[...] = 2.0 * jnp.dot(S, S, preferred_element_type=F32) - eye


def _node_weights_kernel(e_ref, pg_ref, pu_ref, wf_ref, wg_ref, wu_ref,
                         wout_ref):
    Eb = e_ref[...]
    wg_ref[...] = jnp.dot(Eb, pg_ref[...], preferred_element_type=F32)
    wu_ref[...] = jnp.dot(Eb, pu_ref[...], preferred_element_type=F32)
    wout_ref[...] = jnp.dot(Eb, wf_ref[...], preferred_element_type=F32)


def _conv_gate_kernel(s1_ref, s2_ref, fs_ref, fx_ref,
                      t1_ref, t2_ref, tx1_ref, tx2_ref):
    S1 = s1_ref[...]
    S2 = s2_ref[...]
    fs = fs_ref[...]
    fx = fx_ref[...]
    t1_ref[...] = jnp.dot(S1, fs, preferred_element_type=F32)
    t2_ref[...] = jnp.dot(S2, fs, preferred_element_type=F32)
    tx1_ref[...] = jnp.dot(S1, fx, preferred_element_type=F32)
    tx2_ref[...] = jnp.dot(S2, fx, preferred_element_type=F32)


def _conv_cand_kernel(s1_ref, s2_ref, zs_ref, u1_ref, u2_ref):
    S1 = s1_ref[...]
    S2 = s2_ref[...]
    zs = zs_ref[...]
    u1_ref[...] = jnp.dot(S1, zs, preferred_element_type=F32)
    u2_ref[...] = jnp.dot(S2, zs, preferred_element_type=F32)


def _gate_kernel(nblk, b, h, pad, s_ref, t1_ref, t2_ref, xc_ref, wg_ref,
                 zs_ref, r_ref):
    zpad = jnp.zeros((b, pad), F32)
    for i in range(nblk):
        s = s_ref[i]
        feat = jnp.concatenate(
            [s, t1_ref[i], t2_ref[i], xc_ref[i], zpad], axis=1)
        zr = jax.nn.sigmoid(
            jnp.dot(feat, wg_ref[i], preferred_element_type=F32))
        z = zr[:, :h]
        r = zr[:, h:]
        zs_ref[i] = z * s
        r_ref[i] = r


def _cand_kernel(nblk, b, h, pad, zs_ref, u1_ref, u2_ref, xc_ref, r_ref,
                 s_ref, wu_ref, h_ref):
    zpad = jnp.zeros((b, pad), F32)
    for i in range(nblk):
        feat = jnp.concatenate(
            [zs_ref[i], u1_ref[i], u2_ref[i], xc_ref[i], zpad], axis=1)
        hc = jnp.tanh(jnp.dot(feat, wu_ref[i], preferred_element_type=F32))
        r = r_ref[i]
        s = s_ref[i]
        h_ref[:, i, :] = r * s + (1.0 - r) * hc


def kernel(x, state, node_embeddings, gate_w, gate_b, update_w, update_b):
    B, N, C_in = x.shape
    H = state.shape[-1]
    D = node_embeddings.shape[-1]
    K = 3
    C = C_in + H
    KC = K * C
    KCP = -(-KC // 128) * 128
    XCW = K * C_in + 2
    FPAD = KCP - K * H - XCW
    BH = B * H
    BX = B * C_in

    def pool(w, bias, O):
        w3 = w.reshape(D, KC, O).astype(F32)
        parts = [w3[:, k * C + C_in:(k + 1) * C, :] for k in range(K)]
        parts += [w3[:, k * C:k * C + C_in, :] for k in range(K)]
        parts.append(bias[:, None, :].astype(F32))
        parts.append(jnp.zeros((D, KCP - KC - 1, O), F32))
        return jnp.concatenate(parts, axis=1).reshape(D, KCP * O)

    pg = pool(gate_w, gate_b, 2 * H)
    pu = pool(update_w, update_b, H)
    wf = update_w.reshape(D, KC * H).astype(F32)

    sT = jnp.transpose(state, (1, 0, 2))
    xT = jnp.transpose(x, (1, 0, 2))

    par = pltpu.CompilerParams(dimension_semantics=("parallel",),
                               vmem_limit_bytes=64 * 1024 * 1024)

    s1, s2 = pl.pallas_call(
        _supports_kernel,
        out_shape=(jax.ShapeDtypeStruct((N, N), F32),
                   jax.ShapeDtypeStruct((N, N), F32)),
        grid=(1,),
        in_specs=[pl.BlockSpec((N, D), lambda i: (0, 0))],
        out_specs=[pl.BlockSpec((N, N), lambda i: (0, 0)),
                   pl.BlockSpec((N, N), lambda i: (0, 0))],
        compiler_params=pltpu.CompilerParams(
            dimension_semantics=("arbitrary",)),
    )(node_embeddings)

    NBW = 4 if N % 4 == 0 else 1
    nw = N // NBW
    wg2, wu2, wout2 = pl.pallas_call(
        _node_weights_kernel,
        out_shape=(jax.ShapeDtypeStruct((N, KCP * 2 * H), F32),
                   jax.ShapeDtypeStruct((N, KCP * H), F32),
                   jax.ShapeDtypeStruct((N, KC * H), F32)),
        grid=(NBW,),
        in_specs=[pl.BlockSpec((nw, D), lambda i: (i, 0)),
                  pl.BlockSpec((D, KCP * 2 * H), lambda i: (0, 0)),
                  pl.BlockSpec((D, KCP * H), lambda i: (0, 0)),
                  pl.BlockSpec((D, KC * H), lambda i: (0, 0))],
        out_specs=[pl.BlockSpec((nw, KCP * 2 * H), lambda i: (i, 0)),
                   pl.BlockSpec((nw, KCP * H), lambda i: (i, 0)),
                   pl.BlockSpec((nw, KC * H), lambda i: (i, 0))],
        compiler_params=par,
    )(node_embeddings, pg, pu, wf)
    w_out = wout2.reshape(N, K, C, H)
    wg3 = wg2.reshape(N, KCP, 2 * H)
    wu3 = wu2.reshape(N, KCP, H)

    NCH = 8
    fs2 = sT.reshape(N, BH)
    fx2 = xT.reshape(N, BX)
    t1, t2, tx1, tx2 = pl.pallas_call(
        _conv_gate_kernel,
        out_shape=(jax.ShapeDtypeStruct((N, BH), F32),
                   jax.ShapeDtypeStruct((N, BH), F32),
                   jax.ShapeDtypeStruct((N, BX), F32),
                   jax.ShapeDtypeStruct((N, BX), F32)),
        grid=(NCH,),
        in_specs=[pl.BlockSpec((N, N), lambda j: (0, 0)),
                  pl.BlockSpec((N, N), lambda j: (0, 0)),
                  pl.BlockSpec((N, BH // NCH), lambda j: (0, j)),
                  pl.BlockSpec((N, BX // NCH), lambda j: (0, j))],
        out_specs=[pl.BlockSpec((N, BH // NCH), lambda j: (0, j)),
                   pl.BlockSpec((N, BH // NCH), lambda j: (0, j)),
                   pl.BlockSpec((N, BX // NCH), lambda j: (0, j)),
                   pl.BlockSpec((N, BX // NCH), lambda j: (0, j))],
        compiler_params=par,
    )(s1, s2, fs2, fx2)

    xc = jnp.concatenate(
        [xT, tx1.reshape(N, B, C_in), tx2.reshape(N, B, C_in),
         jnp.ones((N, B, 1), F32), jnp.zeros((N, B, 1), F32)], axis=-1)

    NBLK = 8 if N % 8 == 0 else 1
    G = N // NBLK
    t13 = t1.reshape(N, B, H)
    t23 = t2.reshape(N, B, H)
    zs3, r3 = pl.pallas_call(
        functools.partial(_gate_kernel, NBLK, B, H, FPAD),
        out_shape=(jax.ShapeDtypeStruct((N, B, H), F32),
                   jax.ShapeDtypeStruct((N, B, H), F32)),
        grid=(G,),
        in_specs=[pl.BlockSpec((NBLK, B, H), lambda j: (j, 0, 0)),
                  pl.BlockSpec((NBLK, B, H), lambda j: (j, 0, 0)),
                  pl.BlockSpec((NBLK, B, H), lambda j: (j, 0, 0)),
                  pl.BlockSpec((NBLK, B, XCW), lambda j: (j, 0, 0)),
                  pl.BlockSpec((NBLK, KCP, 2 * H), lambda j: (j, 0, 0))],
        out_specs=[pl.BlockSpec((NBLK, B, H), lambda j: (j, 0, 0)),
                   pl.BlockSpec((NBLK, B, H), lambda j: (j, 0, 0))],
        compiler_params=par,
    )(sT, t13, t23, xc, wg3)

    zs2 = zs3.reshape(N, BH)
    u1, u2 = pl.pallas_call(
        _conv_cand_kernel,
        out_shape=(jax.ShapeDtypeStruct((N, BH), F32),
                   jax.ShapeDtypeStruct((N, BH), F32)),
        grid=(NCH,),
        in_specs=[pl.BlockSpec((N, N), lambda j: (0, 0)),
                  pl.BlockSpec((N, N), lambda j: (0, 0)),
                  pl.BlockSpec((N, BH // NCH), lambda j: (0, j))],
        out_specs=[pl.BlockSpec((N, BH // NCH), lambda j: (0, j)),
                   pl.BlockSpec((N, BH // NCH), lambda j: (0, j))],
        compiler_params=par,
    )(s1, s2, zs2)

    u13 = u1.reshape(N, B, H)
    u23 = u2.reshape(N, B, H)
    h = pl.pallas_call(
        functools.partial(_cand_kernel, NBLK, B, H, FPAD),
        out_shape=jax.ShapeDtypeStruct((B, N, H), F32),
        grid=(G,),
        in_specs=[pl.BlockSpec((NBLK, B, H), lambda j: (j, 0, 0)),
                  pl.BlockSpec((NBLK, B, H), lambda j: (j, 0, 0)),
                  pl.BlockSpec((NBLK, B, H), lambda j: (j, 0, 0)),
                  pl.BlockSpec((NBLK, B, XCW), lambda j: (j, 0, 0)),
                  pl.BlockSpec((NBLK, B, H), lambda j: (j, 0, 0)),
                  pl.BlockSpec((NBLK, B, H), lambda j: (j, 0, 0)),
                  pl.BlockSpec((NBLK, KCP, H), lambda j: (j, 0, 0))],
        out_specs=pl.BlockSpec((B, NBLK, H), lambda j: (0, j, 0)),
        compiler_params=par,
    )(zs3, u13, u23, xc, r3, sT, wu3)

    return h, w_out


# speedup vs baseline: 3.9869x; 3.9869x over previous
import functools

import jax
import jax.numpy as jnp
from jax import lax
from jax.experimental import pallas as pl
from jax.experimental.pallas import tpu as pltpu

F32 = jnp.float32
BF16 = jnp.bfloat16


def _supports_kernel(c_in, e_ref, fx_ref, s1_ref, s2_ref, xc_ref):
    E = e_ref[...]
    A = lax.dot_general(E, E, (((1,), (1,)), ((), ())),
                        preferred_element_type=F32)
    A = jnp.maximum(A, 0.0)
    A = A - jnp.max(A, axis=1, keepdims=True)
    eA = jnp.exp(A)
    S = eA / jnp.sum(eA, axis=1, keepdims=True)
    n = S.shape[0]
    row = lax.broadcasted_iota(jnp.int32, (n, n), 0)
    col = lax.broadcasted_iota(jnp.int32, (n, n), 1)
    eye = (row == col).astype(F32)
    T2 = 2.0 * jnp.dot(S, S, preferred_element_type=F32) - eye
    s1_ref[...] = S
    s2_ref[...] = T2
    b = fx_ref.shape[-1]
    for c in range(c_in):
        xr = fx_ref[:, c, :].astype(F32)
        xc_ref[:, c, :] = fx_ref[:, c, :]
        xc_ref[:, c_in + c, :] = jnp.dot(
            S, xr, preferred_element_type=F32).astype(BF16)
        xc_ref[:, 2 * c_in + c, :] = jnp.dot(
            T2, xr, preferred_element_type=F32).astype(BF16)
    xc_ref[:, 3 * c_in, :] = jnp.ones((n, b), BF16)
    xc_ref[:, 3 * c_in + 1, :] = jnp.zeros((n, b), BF16)


def _node_weights_kernel(e_ref, pg_ref, pu_ref, wf_ref, wg_ref, wu_ref,
                         wout_ref):
    Eb = e_ref[...]
    wg_ref[...] = jnp.dot(Eb, pg_ref[...],
                          preferred_element_type=F32).astype(BF16)
    wu_ref[...] = jnp.dot(Eb, pu_ref[...],
                          preferred_element_type=F32).astype(BF16)
    wout_ref[...] = jnp.dot(Eb, wf_ref[...], preferred_element_type=F32)


def _conv_gate_kernel(hc, s1_ref, s2_ref, fs_ref, t1_ref, t2_ref):
    S1 = s1_ref[...]
    S2 = s2_ref[...]
    for i in range(hc):
        r = fs_ref[:, i, :]
        t1_ref[:, i, :] = jnp.dot(S1, r, preferred_element_type=F32
                                  ).astype(BF16)
        t2_ref[:, i, :] = jnp.dot(S2, r, preferred_element_type=F32
                                  ).astype(BF16)


def _conv_cand_kernel(hc, s1_ref, s2_ref, zs_ref, u1_ref, u2_ref):
    S1 = s1_ref[...].astype(BF16)
    S2 = s2_ref[...].astype(BF16)
    for i in range(hc):
        r = zs_ref[:, i, :]
        u1_ref[:, i, :] = jnp.dot(S1, r, preferred_element_type=F32
                                  ).astype(BF16)
        u2_ref[:, i, :] = jnp.dot(S2, r, preferred_element_type=F32
                                  ).astype(BF16)


def _gate_kernel(nblk, b, h, pad, s_ref, t1_ref, t2_ref, xc_ref, wg_ref,
                 zs_ref, r_ref):
    zpad = jnp.zeros((pad, b), BF16)
    for i in range(nblk):
        s = s_ref[i]
        feat = jnp.concatenate(
            [s.astype(BF16), t1_ref[i], t2_ref[i], xc_ref[i], zpad], axis=0)
        zr = jax.nn.sigmoid(
            lax.dot_general(wg_ref[i], feat, (((0,), (0,)), ((), ())),
                            preferred_element_type=F32))
        z = zr[:h]
        r = zr[h:]
        zs_ref[i] = (z * s).astype(BF16)
        r_ref[i] = r


def _cand_kernel(nblk, b, h, pad, zs_ref, u1_ref, u2_ref, xc_ref, r_ref,
                 s_ref, wu_ref, h_ref):
    zpad = jnp.zeros((pad, b), BF16)
    for i in range(nblk):
        feat = jnp.concatenate(
            [zs_ref[i], u1_ref[i], u2_ref[i], xc_ref[i], zpad], axis=0)
        hc = jnp.tanh(
            lax.dot_general(wu_ref[i], feat, (((0,), (0,)), ((), ())),
                            preferred_element_type=F32))
        r = r_ref[i]
        s = s_ref[i]
        h_ref[i] = r * s + (1.0 - r) * hc


def kernel(x, state, node_embeddings, gate_w, gate_b, update_w, update_b):
    B, N, C_in = x.shape
    H = state.shape[-1]
    D = node_embeddings.shape[-1]
    K = 3
    C = C_in + H
    KC = K * C
    KCP = -(-KC // 128) * 128
    XCW = K * C_in + 2
    FPAD = KCP - K * H - XCW

    def pool(w, bias, O):
        w3 = w.reshape(D, KC, O).astype(F32)
        parts = [w3[:, k * C + C_in:(k + 1) * C, :] for k in range(K)]
        parts += [w3[:, k * C:k * C + C_in, :] for k in range(K)]
        parts.append(bias[:, None, :].astype(F32))
        parts.append(jnp.zeros((D, KCP - KC - 1, O), F32))
        return jnp.concatenate(parts, axis=1).reshape(D, KCP * O)

    pg = pool(gate_w, gate_b, 2 * H)
    pu = pool(update_w, update_b, H)
    wf = update_w.reshape(D, KC * H).astype(F32)

    sT = jnp.transpose(state, (1, 2, 0))
    xT = jnp.transpose(x, (1, 2, 0)).astype(BF16)

    par = pltpu.CompilerParams(dimension_semantics=("parallel",),
                               vmem_limit_bytes=64 * 1024 * 1024)

    s1, s2, xc = pl.pallas_call(
        functools.partial(_supports_kernel, C_in),
        out_shape=(jax.ShapeDtypeStruct((N, N), F32),
                   jax.ShapeDtypeStruct((N, N), F32),
                   jax.ShapeDtypeStruct((N, XCW, B), BF16)),
        grid=(1,),
        in_specs=[pl.BlockSpec((N, D), lambda i: (0, 0)),
                  pl.BlockSpec((N, C_in, B), lambda i: (0, 0, 0))],
        out_specs=[pl.BlockSpec((N, N), lambda i: (0, 0)),
                   pl.BlockSpec((N, N), lambda i: (0, 0)),
                   pl.BlockSpec((N, XCW, B), lambda i: (0, 0, 0))],
        compiler_params=pltpu.CompilerParams(
            dimension_semantics=("arbitrary",)),
    )(node_embeddings, xT)

    NBW = 4 if N % 4 == 0 else 1
    nw = N // NBW
    wg2, wu2, wout2 = pl.pallas_call(
        _node_weights_kernel,
        out_shape=(jax.ShapeDtypeStruct((N, KCP * 2 * H), BF16),
                   jax.ShapeDtypeStruct((N, KCP * H), BF16),
                   jax.ShapeDtypeStruct((N, KC * H), F32)),
        grid=(NBW,),
        in_specs=[pl.BlockSpec((nw, D), lambda i: (i, 0)),
                  pl.BlockSpec((D, KCP * 2 * H), lambda i: (0, 0)),
                  pl.BlockSpec((D, KCP * H), lambda i: (0, 0)),
                  pl.BlockSpec((D, KC * H), lambda i: (0, 0))],
        out_specs=[pl.BlockSpec((nw, KCP * 2 * H), lambda i: (i, 0)),
                   pl.BlockSpec((nw, KCP * H), lambda i: (i, 0)),
                   pl.BlockSpec((nw, KC * H), lambda i: (i, 0))],
        compiler_params=par,
    )(node_embeddings, pg, pu, wf)
    w_out = wout2.reshape(N, K, C, H)
    wg3 = wg2.reshape(N, KCP, 2 * H)
    wu3 = wu2.reshape(N, KCP, H)

    NCH = 8
    hc = H // NCH
    t1, t2 = pl.pallas_call(
        functools.partial(_conv_gate_kernel, hc),
        out_shape=(jax.ShapeDtypeStruct((N, H, B), BF16),
                   jax.ShapeDtypeStruct((N, H, B), BF16)),
        grid=(NCH,),
        in_specs=[pl.BlockSpec((N, N), lambda j: (0, 0)),
                  pl.BlockSpec((N, N), lambda j: (0, 0)),
                  pl.BlockSpec((N, hc, B), lambda j: (0, j, 0))],
        out_specs=[pl.BlockSpec((N, hc, B), lambda j: (0, j, 0)),
                   pl.BlockSpec((N, hc, B), lambda j: (0, j, 0))],
        compiler_params=par,
    )(s1, s2, sT)

    NBLK = 8 if N % 8 == 0 else 1
    G = N // NBLK
    zs, r3 = pl.pallas_call(
        functools.partial(_gate_kernel, NBLK, B, H, FPAD),
        out_shape=(jax.ShapeDtypeStruct((N, H, B), BF16),
                   jax.ShapeDtypeStruct((N, H, B), F32)),
        grid=(G,),
        in_specs=[pl.BlockSpec((NBLK, H, B), lambda j: (j, 0, 0)),
                  pl.BlockSpec((NBLK, H, B), lambda j: (j, 0, 0)),
                  pl.BlockSpec((NBLK, H, B), lambda j: (j, 0, 0)),
                  pl.BlockSpec((NBLK, XCW, B), lambda j: (j, 0, 0)),
                  pl.BlockSpec((NBLK, KCP, 2 * H), lambda j: (j, 0, 0))],
        out_specs=[pl.BlockSpec((NBLK, H, B), lambda j: (j, 0, 0)),
                   pl.BlockSpec((NBLK, H, B), lambda j: (j, 0, 0))],
        compiler_params=par,
    )(sT, t1, t2, xc, wg3)

    u1, u2 = pl.pallas_call(
        functools.partial(_conv_cand_kernel, hc),
        out_shape=(jax.ShapeDtypeStruct((N, H, B), BF16),
                   jax.ShapeDtypeStruct((N, H, B), BF16)),
        grid=(NCH,),
        in_specs=[pl.BlockSpec((N, N), lambda j: (0, 0)),
                  pl.BlockSpec((N, N), lambda j: (0, 0)),
                  pl.BlockSpec((N, hc, B), lambda j: (0, j, 0))],
        out_specs=[pl.BlockSpec((N, hc, B), lambda j: (0, j, 0)),
                   pl.BlockSpec((N, hc, B), lambda j: (0, j, 0))],
        compiler_params=par,
    )(s1, s2, zs)

    hb = pl.pallas_call(
        functools.partial(_cand_kernel, NBLK, B, H, FPAD),
        out_shape=jax.ShapeDtypeStruct((N, H, B), F32),
        grid=(G,),
        in_specs=[pl.BlockSpec((NBLK, H, B), lambda j: (j, 0, 0)),
                  pl.BlockSpec((NBLK, H, B), lambda j: (j, 0, 0)),
                  pl.BlockSpec((NBLK, H, B), lambda j: (j, 0, 0)),
                  pl.BlockSpec((NBLK, XCW, B), lambda j: (j, 0, 0)),
                  pl.BlockSpec((NBLK, H, B), lambda j: (j, 0, 0)),
                  pl.BlockSpec((NBLK, H, B), lambda j: (j, 0, 0)),
                  pl.BlockSpec((NBLK, KCP, H), lambda j: (j, 0, 0))],
        out_specs=pl.BlockSpec((NBLK, H, B), lambda j: (j, 0, 0)),
        compiler_params=par,
    )(zs, u1, u2, xc, r3, sT, wu3)

    h = jnp.transpose(hb, (2, 0, 1))
    return h, w_out


# speedup vs baseline: 4.2048x; 1.0546x over previous
import functools

import jax
import jax.numpy as jnp
from jax import lax
from jax.experimental import pallas as pl
from jax.experimental.pallas import tpu as pltpu

F32 = jnp.float32
BF16 = jnp.bfloat16


def _supports_kernel(c_in, e_ref, fx_ref, s1_ref, s2_ref, xc_ref):
    E = e_ref[...]
    A = lax.dot_general(E, E, (((1,), (1,)), ((), ())),
                        preferred_element_type=F32)
    A = jnp.maximum(A, 0.0)
    A = A - jnp.max(A, axis=1, keepdims=True)
    eA = jnp.exp(A)
    S = eA / jnp.sum(eA, axis=1, keepdims=True)
    n = S.shape[0]
    row = lax.broadcasted_iota(jnp.int32, (n, n), 0)
    col = lax.broadcasted_iota(jnp.int32, (n, n), 1)
    eye = (row == col).astype(F32)
    T2 = 2.0 * jnp.dot(S, S, preferred_element_type=F32) - eye
    s1_ref[...] = S
    s2_ref[...] = T2
    b = fx_ref.shape[-1]
    for c in range(c_in):
        xr = fx_ref[:, c, :].astype(F32)
        xc_ref[:, c, :] = fx_ref[:, c, :]
        xc_ref[:, c_in + c, :] = jnp.dot(
            S, xr, preferred_element_type=F32).astype(BF16)
        xc_ref[:, 2 * c_in + c, :] = jnp.dot(
            T2, xr, preferred_element_type=F32).astype(BF16)
    xc_ref[:, 3 * c_in, :] = jnp.ones((n, b), BF16)
    xc_ref[:, 3 * c_in + 1, :] = jnp.zeros((n, b), BF16)


def _node_weights_kernel(e_ref, pg_ref, pu_ref, wf_ref, wg_ref, wu_ref,
                         wout_ref):
    Eb = e_ref[...]
    wg_ref[...] = jnp.dot(Eb, pg_ref[...],
                          preferred_element_type=F32).astype(BF16)
    wu_ref[...] = jnp.dot(Eb, pu_ref[...],
                          preferred_element_type=F32).astype(BF16)
    wout_ref[...] = jnp.dot(Eb, wf_ref[...], preferred_element_type=F32)


def _conv_gate_kernel(hc, s1_ref, s2_ref, fs_ref, t1_ref, t2_ref):
    S1 = s1_ref[...]
    S2 = s2_ref[...]
    for i in range(hc):
        r = fs_ref[:, i, :]
        t1_ref[:, i, :] = jnp.dot(S1, r, preferred_element_type=F32
                                  ).astype(BF16)
        t2_ref[:, i, :] = jnp.dot(S2, r, preferred_element_type=F32
                                  ).astype(BF16)


def _conv_cand_kernel(hc, s1_ref, s2_ref, zs_ref, u1_ref, u2_ref):
    S1 = s1_ref[...]
    S2 = s2_ref[...]
    for i in range(hc):
        r = zs_ref[:, i, :]
        u1_ref[:, i, :] = jnp.dot(S1, r, preferred_element_type=F32
                                  ).astype(BF16)
        u2_ref[:, i, :] = jnp.dot(S2, r, preferred_element_type=F32
                                  ).astype(BF16)


def _gate_kernel(nblk, b, h, pad, s_ref, t1_ref, t2_ref, xc_ref, wg_ref,
                 zs_ref, r_ref):
    zpad = jnp.zeros((pad, b), BF16)
    for i in range(nblk):
        s = s_ref[i]
        feat = jnp.concatenate(
            [s.astype(BF16), t1_ref[i], t2_ref[i], xc_ref[i], zpad], axis=0)
        zr = jax.nn.sigmoid(
            lax.dot_general(wg_ref[i], feat, (((0,), (0,)), ((), ())),
                            preferred_element_type=F32))
        z = zr[:h]
        r = zr[h:]
        zs_ref[i] = z * s
        r_ref[i] = r


def _cand_kernel(nblk, b, h, pad, zs_ref, u1_ref, u2_ref, xc_ref, r_ref,
                 s_ref, wu_ref, h_ref):
    zpad = jnp.zeros((pad, b), BF16)
    for i in range(nblk):
        feat = jnp.concatenate(
            [zs_ref[i].astype(BF16), u1_ref[i], u2_ref[i], xc_ref[i], zpad],
            axis=0)
        hc = jnp.tanh(
            lax.dot_general(wu_ref[i], feat, (((0,), (0,)), ((), ())),
                            preferred_element_type=F32))
        r = r_ref[i]
        s = s_ref[i]
        h_ref[i] = r * s + (1.0 - r) * hc


def kernel(x, state, node_embeddings, gate_w, gate_b, update_w, update_b):
    B, N, C_in = x.shape
    H = state.shape[-1]
    D = node_embeddings.shape[-1]
    K = 3
    C = C_in + H
    KC = K * C
    KCP = -(-KC // 128) * 128
    XCW = K * C_in + 2
    FPAD = KCP - K * H - XCW

    def pool(w, bias, O):
        w3 = w.reshape(D, KC, O).astype(F32)
        parts = [w3[:, k * C + C_in:(k + 1) * C, :] for k in range(K)]
        parts += [w3[:, k * C:k * C + C_in, :] for k in range(K)]
        parts.append(bias[:, None, :].astype(F32))
        parts.append(jnp.zeros((D, KCP - KC - 1, O), F32))
        return jnp.concatenate(parts, axis=1).reshape(D, KCP * O)

    pg = pool(gate_w, gate_b, 2 * H)
    pu = pool(update_w, update_b, H)
    wf = update_w.reshape(D, KC * H).astype(F32)

    sT = jnp.transpose(state, (1, 2, 0))
    xT = jnp.transpose(x, (1, 2, 0)).astype(BF16)

    par = pltpu.CompilerParams(dimension_semantics=("parallel",),
                               vmem_limit_bytes=64 * 1024 * 1024)

    s1, s2, xc = pl.pallas_call(
        functools.partial(_supports_kernel, C_in),
        out_shape=(jax.ShapeDtypeStruct((N, N), F32),
                   jax.ShapeDtypeStruct((N, N), F32),
                   jax.ShapeDtypeStruct((N, XCW, B), BF16)),
        grid=(1,),
        in_specs=[pl.BlockSpec((N, D), lambda i: (0, 0)),
                  pl.BlockSpec((N, C_in, B), lambda i: (0, 0, 0))],
        out_specs=[pl.BlockSpec((N, N), lambda i: (0, 0)),
                   pl.BlockSpec((N, N), lambda i: (0, 0)),
                   pl.BlockSpec((N, XCW, B), lambda i: (0, 0, 0))],
        compiler_params=pltpu.CompilerParams(
            dimension_semantics=("arbitrary",)),
    )(node_embeddings, xT)

    NBW = 4 if N % 4 == 0 else 1
    nw = N // NBW
    wg2, wu2, wout2 = pl.pallas_call(
        _node_weights_kernel,
        out_shape=(jax.ShapeDtypeStruct((N, KCP * 2 * H), BF16),
                   jax.ShapeDtypeStruct((N, KCP * H), BF16),
                   jax.ShapeDtypeStruct((N, KC * H), F32)),
        grid=(NBW,),
        in_specs=[pl.BlockSpec((nw, D), lambda i: (i, 0)),
                  pl.BlockSpec((D, KCP * 2 * H), lambda i: (0, 0)),
                  pl.BlockSpec((D, KCP * H), lambda i: (0, 0)),
                  pl.BlockSpec((D, KC * H), lambda i: (0, 0))],
        out_specs=[pl.BlockSpec((nw, KCP * 2 * H), lambda i: (i, 0)),
                   pl.BlockSpec((nw, KCP * H), lambda i: (i, 0)),
                   pl.BlockSpec((nw, KC * H), lambda i: (i, 0))],
        compiler_params=par,
    )(node_embeddings, pg, pu, wf)
    w_out = wout2.reshape(N, K, C, H)
    wg3 = wg2.reshape(N, KCP, 2 * H)
    wu3 = wu2.reshape(N, KCP, H)

    NCH = 8
    hc = H // NCH
    t1, t2 = pl.pallas_call(
        functools.partial(_conv_gate_kernel, hc),
        out_shape=(jax.ShapeDtypeStruct((N, H, B), BF16),
                   jax.ShapeDtypeStruct((N, H, B), BF16)),
        grid=(NCH,),
        in_specs=[pl.BlockSpec((N, N), lambda j: (0, 0)),
                  pl.BlockSpec((N, N), lambda j: (0, 0)),
                  pl.BlockSpec((N, hc, B), lambda j: (0, j, 0))],
        out_specs=[pl.BlockSpec((N, hc, B), lambda j: (0, j, 0)),
                   pl.BlockSpec((N, hc, B), lambda j: (0, j, 0))],
        compiler_params=par,
    )(s1, s2, sT)

    NBLK = 8 if N % 8 == 0 else 1
    G = N // NBLK
    zs, r3 = pl.pallas_call(
        functools.partial(_gate_kernel, NBLK, B, H, FPAD),
        out_shape=(jax.ShapeDtypeStruct((N, H, B), F32),
                   jax.ShapeDtypeStruct((N, H, B), F32)),
        grid=(G,),
        in_specs=[pl.BlockSpec((NBLK, H, B), lambda j: (j, 0, 0)),
                  pl.BlockSpec((NBLK, H, B), lambda j: (j, 0, 0)),
                  pl.BlockSpec((NBLK, H, B), lambda j: (j, 0, 0)),
                  pl.BlockSpec((NBLK, XCW, B), lambda j: (j, 0, 0)),
                  pl.BlockSpec((NBLK, KCP, 2 * H), lambda j: (j, 0, 0))],
        out_specs=[pl.BlockSpec((NBLK, H, B), lambda j: (j, 0, 0)),
                   pl.BlockSpec((NBLK, H, B), lambda j: (j, 0, 0))],
        compiler_params=par,
    )(sT, t1, t2, xc, wg3)

    u1, u2 = pl.pallas_call(
        functools.partial(_conv_cand_kernel, hc),
        out_shape=(jax.ShapeDtypeStruct((N, H, B), BF16),
                   jax.ShapeDtypeStruct((N, H, B), BF16)),
        grid=(NCH,),
        in_specs=[pl.BlockSpec((N, N), lambda j: (0, 0)),
                  pl.BlockSpec((N, N), lambda j: (0, 0)),
                  pl.BlockSpec((N, hc, B), lambda j: (0, j, 0))],
        out_specs=[pl.BlockSpec((N, hc, B), lambda j: (0, j, 0)),
                   pl.BlockSpec((N, hc, B), lambda j: (0, j, 0))],
        compiler_params=par,
    )(s1, s2, zs)

    hb = pl.pallas_call(
        functools.partial(_cand_kernel, NBLK, B, H, FPAD),
        out_shape=jax.ShapeDtypeStruct((N, H, B), F32),
        grid=(G,),
        in_specs=[pl.BlockSpec((NBLK, H, B), lambda j: (j, 0, 0)),
                  pl.BlockSpec((NBLK, H, B), lambda j: (j, 0, 0)),
                  pl.BlockSpec((NBLK, H, B), lambda j: (j, 0, 0)),
                  pl.BlockSpec((NBLK, XCW, B), lambda j: (j, 0, 0)),
                  pl.BlockSpec((NBLK, H, B), lambda j: (j, 0, 0)),
                  pl.BlockSpec((NBLK, H, B), lambda j: (j, 0, 0)),
                  pl.BlockSpec((NBLK, KCP, H), lambda j: (j, 0, 0))],
        out_specs=pl.BlockSpec((NBLK, H, B), lambda j: (j, 0, 0)),
        compiler_params=par,
    )(zs, u1, u2, xc, r3, sT, wu3)

    h = jnp.transpose(hb, (2, 0, 1))
    return h, w_out


# speedup vs baseline: 4.2964x; 1.0218x over previous
import functools

import jax
import jax.numpy as jnp
from jax import lax
from jax.experimental import pallas as pl
from jax.experimental.pallas import tpu as pltpu

F32 = jnp.float32
BF16 = jnp.bfloat16


def _supports_kernel(c_in, e_ref, fx_ref, s12_ref, xc_ref):
    E = e_ref[...]
    A = lax.dot_general(E, E, (((1,), (1,)), ((), ())),
                        preferred_element_type=F32)
    A = jnp.maximum(A, 0.0)
    A = A - jnp.max(A, axis=1, keepdims=True)
    eA = jnp.exp(A)
    S = eA / jnp.sum(eA, axis=1, keepdims=True)
    n = S.shape[0]
    row = lax.broadcasted_iota(jnp.int32, (n, n), 0)
    col = lax.broadcasted_iota(jnp.int32, (n, n), 1)
    eye = (row == col).astype(F32)
    T2 = 2.0 * jnp.dot(S, S, preferred_element_type=F32) - eye
    s12_ref[...] = jnp.concatenate([S, T2], axis=0)
    b = fx_ref.shape[-1]
    S12 = jnp.concatenate([S, T2], axis=0)
    for c in range(c_in):
        xr = fx_ref[:, c, :].astype(F32)
        tx = jnp.dot(S12, xr, preferred_element_type=F32).astype(BF16)
        xc_ref[:, c, :] = fx_ref[:, c, :]
        xc_ref[:, c_in + c, :] = tx[:n]
        xc_ref[:, 2 * c_in + c, :] = tx[n:]
    xc_ref[:, 3 * c_in, :] = jnp.ones((n, b), BF16)
    xc_ref[:, 3 * c_in + 1, :] = jnp.zeros((n, b), BF16)


def _node_weights_kernel(e_ref, pg_ref, pu_ref, wf_ref, wg_ref, wu_ref,
                         wout_ref):
    Eb = e_ref[...]
    wg_ref[...] = jnp.dot(Eb, pg_ref[...],
                          preferred_element_type=F32).astype(BF16)
    wu_ref[...] = jnp.dot(Eb, pu_ref[...],
                          preferred_element_type=F32).astype(BF16)
    wout_ref[...] = jnp.dot(Eb, wf_ref[...], preferred_element_type=F32)


def _conv_kernel(hc, n, s12_ref, fs_ref, t1_ref, t2_ref):
    S12 = s12_ref[...]
    for i in range(hc):
        r = fs_ref[:, i, :]
        t = jnp.dot(S12, r, preferred_element_type=F32).astype(BF16)
        t1_ref[:, i, :] = t[:n]
        t2_ref[:, i, :] = t[n:]


def _gate_kernel(nblk, b, h, pad, s_ref, t1_ref, t2_ref, xc_ref, wg_ref,
                 zs_ref, r_ref):
    zpad = jnp.zeros((pad, b), BF16)
    for i in range(nblk):
        s = s_ref[i]
        feat = jnp.concatenate(
            [s.astype(BF16), t1_ref[i], t2_ref[i], xc_ref[i], zpad], axis=0)
        zr = jax.nn.sigmoid(
            lax.dot_general(wg_ref[i], feat, (((0,), (0,)), ((), ())),
                            preferred_element_type=F32))
        z = zr[:h]
        r = zr[h:]
        zs_ref[i] = z * s
        r_ref[i] = r.astype(BF16)


def _cand_kernel(nblk, b, h, pad, zs_ref, u1_ref, u2_ref, xc_ref, r_ref,
                 s_ref, wu_ref, h_ref):
    zpad = jnp.zeros((pad, b), BF16)
    for i in range(nblk):
        feat = jnp.concatenate(
            [zs_ref[i].astype(BF16), u1_ref[i], u2_ref[i], xc_ref[i], zpad],
            axis=0)
        hc = jnp.tanh(
            lax.dot_general(wu_ref[i], feat, (((0,), (0,)), ((), ())),
                            preferred_element_type=F32))
        r = r_ref[i].astype(F32)
        s = s_ref[i]
        h_ref[i] = r * s + (1.0 - r) * hc


def kernel(x, state, node_embeddings, gate_w, gate_b, update_w, update_b):
    B, N, C_in = x.shape
    H = state.shape[-1]
    D = node_embeddings.shape[-1]
    K = 3
    C = C_in + H
    KC = K * C
    KCP = -(-KC // 128) * 128
    XCW = K * C_in + 2
    FPAD = KCP - K * H - XCW

    def pool(w, bias, O):
        w3 = w.reshape(D, KC, O).astype(F32)
        parts = [w3[:, k * C + C_in:(k + 1) * C, :] for k in range(K)]
        parts += [w3[:, k * C:k * C + C_in, :] for k in range(K)]
        parts.append(bias[:, None, :].astype(F32))
        parts.append(jnp.zeros((D, KCP - KC - 1, O), F32))
        return jnp.concatenate(parts, axis=1).reshape(D, KCP * O)

    pg = pool(gate_w, gate_b, 2 * H)
    pu = pool(update_w, update_b, H)
    wf = update_w.reshape(D, KC * H).astype(F32)

    sT = jnp.transpose(state, (1, 2, 0))
    xT = jnp.transpose(x, (1, 2, 0)).astype(BF16)

    par = pltpu.CompilerParams(dimension_semantics=("parallel",),
                               vmem_limit_bytes=64 * 1024 * 1024)

    s12, xc = pl.pallas_call(
        functools.partial(_supports_kernel, C_in),
        out_shape=(jax.ShapeDtypeStruct((2 * N, N), F32),
                   jax.ShapeDtypeStruct((N, XCW, B), BF16)),
        grid=(1,),
        in_specs=[pl.BlockSpec((N, D), lambda i: (0, 0)),
                  pl.BlockSpec((N, C_in, B), lambda i: (0, 0, 0))],
        out_specs=[pl.BlockSpec((2 * N, N), lambda i: (0, 0)),
                   pl.BlockSpec((N, XCW, B), lambda i: (0, 0, 0))],
        compiler_params=pltpu.CompilerParams(
            dimension_semantics=("arbitrary",)),
    )(node_embeddings, xT)

    NBW = 4 if N % 4 == 0 else 1
    nw = N // NBW
    wg2, wu2, wout2 = pl.pallas_call(
        _node_weights_kernel,
        out_shape=(jax.ShapeDtypeStruct((N, KCP * 2 * H), BF16),
                   jax.ShapeDtypeStruct((N, KCP * H), BF16),
                   jax.ShapeDtypeStruct((N, KC * H), F32)),
        grid=(NBW,),
        in_specs=[pl.BlockSpec((nw, D), lambda i: (i, 0)),
                  pl.BlockSpec((D, KCP * 2 * H), lambda i: (0, 0)),
                  pl.BlockSpec((D, KCP * H), lambda i: (0, 0)),
                  pl.BlockSpec((D, KC * H), lambda i: (0, 0))],
        out_specs=[pl.BlockSpec((nw, KCP * 2 * H), lambda i: (i, 0)),
                   pl.BlockSpec((nw, KCP * H), lambda i: (i, 0)),
                   pl.BlockSpec((nw, KC * H), lambda i: (i, 0))],
        compiler_params=par,
    )(node_embeddings, pg, pu, wf)
    w_out = wout2.reshape(N, K, C, H)
    wg3 = wg2.reshape(N, KCP, 2 * H)
    wu3 = wu2.reshape(N, KCP, H)

    NCH = 8
    hc = H // NCH
    t1, t2 = pl.pallas_call(
        functools.partial(_conv_kernel, hc, N),
        out_shape=(jax.ShapeDtypeStruct((N, H, B), BF16),
                   jax.ShapeDtypeStruct((N, H, B), BF16)),
        grid=(NCH,),
        in_specs=[pl.BlockSpec((2 * N, N), lambda j: (0, 0)),
                  pl.BlockSpec((N, hc, B), lambda j: (0, j, 0))],
        out_specs=[pl.BlockSpec((N, hc, B), lambda j: (0, j, 0)),
                   pl.BlockSpec((N, hc, B), lambda j: (0, j, 0))],
        compiler_params=par,
    )(s12, sT)

    NBLK = 8 if N % 8 == 0 else 1
    G = N // NBLK
    zs, r3 = pl.pallas_call(
        functools.partial(_gate_kernel, NBLK, B, H, FPAD),
        out_shape=(jax.ShapeDtypeStruct((N, H, B), F32),
                   jax.ShapeDtypeStruct((N, H, B), BF16)),
        grid=(G,),
        in_specs=[pl.BlockSpec((NBLK, H, B), lambda j: (j, 0, 0)),
                  pl.BlockSpec((NBLK, H, B), lambda j: (j, 0, 0)),
                  pl.BlockSpec((NBLK, H, B), lambda j: (j, 0, 0)),
                  pl.BlockSpec((NBLK, XCW, B), lambda j: (j, 0, 0)),
                  pl.BlockSpec((NBLK, KCP, 2 * H), lambda j: (j, 0, 0))],
        out_specs=[pl.BlockSpec((NBLK, H, B), lambda j: (j, 0, 0)),
                   pl.BlockSpec((NBLK, H, B), lambda j: (j, 0, 0))],
        compiler_params=par,
    )(sT, t1, t2, xc, wg3)

    u1, u2 = pl.pallas_call(
        functools.partial(_conv_kernel, hc, N),
        out_shape=(jax.ShapeDtypeStruct((N, H, B), BF16),
                   jax.ShapeDtypeStruct((N, H, B), BF16)),
        grid=(NCH,),
        in_specs=[pl.BlockSpec((2 * N, N), lambda j: (0, 0)),
                  pl.BlockSpec((N, hc, B), lambda j: (0, j, 0))],
        out_specs=[pl.BlockSpec((N, hc, B), lambda j: (0, j, 0)),
                   pl.BlockSpec((N, hc, B), lambda j: (0, j, 0))],
        compiler_params=par,
    )(s12, zs)

    hb = pl.pallas_call(
        functools.partial(_cand_kernel, NBLK, B, H, FPAD),
        out_shape=jax.ShapeDtypeStruct((N, H, B), F32),
        grid=(G,),
        in_specs=[pl.BlockSpec((NBLK, H, B), lambda j: (j, 0, 0)),
                  pl.BlockSpec((NBLK, H, B), lambda j: (j, 0, 0)),
                  pl.BlockSpec((NBLK, H, B), lambda j: (j, 0, 0)),
                  pl.BlockSpec((NBLK, XCW, B), lambda j: (j, 0, 0)),
                  pl.BlockSpec((NBLK, H, B), lambda j: (j, 0, 0)),
                  pl.BlockSpec((NBLK, H, B), lambda j: (j, 0, 0)),
                  pl.BlockSpec((NBLK, KCP, H), lambda j: (j, 0, 0))],
        out_specs=pl.BlockSpec((NBLK, H, B), lambda j: (j, 0, 0)),
        compiler_params=par,
    )(zs, u1, u2, xc, r3, sT, wu3)

    h = jnp.transpose(hb, (2, 0, 1))
    return h, w_out


# speedup vs baseline: 4.4385x; 1.0331x over previous
import functools

import jax
import jax.numpy as jnp
from jax import lax
from jax.experimental import pallas as pl
from jax.experimental.pallas import tpu as pltpu

F32 = jnp.float32
BF16 = jnp.bfloat16


def _supports_kernel(c_in, e_ref, fx_ref, s12_ref, xc_ref):
    E = e_ref[...]
    A = lax.dot_general(E, E, (((1,), (1,)), ((), ())),
                        preferred_element_type=F32)
    A = jnp.maximum(A, 0.0)
    A = A - jnp.max(A, axis=1, keepdims=True)
    eA = jnp.exp(A)
    S = eA / jnp.sum(eA, axis=1, keepdims=True)
    n = S.shape[0]
    row = lax.broadcasted_iota(jnp.int32, (n, n), 0)
    col = lax.broadcasted_iota(jnp.int32, (n, n), 1)
    eye = (row == col).astype(F32)
    T2 = 2.0 * jnp.dot(S, S, preferred_element_type=F32) - eye
    s12_ref[...] = jnp.concatenate([S, T2], axis=0)
    b = fx_ref.shape[-1]
    S12 = jnp.concatenate([S, T2], axis=0)
    for c in range(c_in):
        xr = fx_ref[:, c, :].astype(F32)
        tx = jnp.dot(S12, xr, preferred_element_type=F32).astype(BF16)
        xc_ref[:, c, :] = fx_ref[:, c, :]
        xc_ref[:, c_in + c, :] = tx[:n]
        xc_ref[:, 2 * c_in + c, :] = tx[n:]
    xc_ref[:, 3 * c_in, :] = jnp.ones((n, b), BF16)
    xc_ref[:, 3 * c_in + 1, :] = jnp.zeros((n, b), BF16)


def _node_weights_kernel(e_ref, pg_ref, pu_ref, wf_ref, wg_ref, wu_ref,
                         wout_ref):
    Eb = e_ref[...]
    wg_ref[...] = jnp.dot(Eb, pg_ref[...],
                          preferred_element_type=F32).astype(BF16)
    wu_ref[...] = jnp.dot(Eb, pu_ref[...],
                          preferred_element_type=F32).astype(BF16)
    wout_ref[...] = jnp.dot(Eb, wf_ref[...], preferred_element_type=F32)


def _conv_kernel(hc, n, s12_ref, fs_ref, t1_ref, t2_ref):
    S12 = s12_ref[...]
    b = fs_ref.shape[-1]
    if hc % 2 == 0:
        for i in range(0, hc, 2):
            r = jnp.concatenate([fs_ref[:, i, :], fs_ref[:, i + 1, :]],
                                axis=1)
            t = jnp.dot(S12, r, preferred_element_type=F32).astype(BF16)
            t1_ref[:, i, :] = t[:n, :b]
            t1_ref[:, i + 1, :] = t[:n, b:]
            t2_ref[:, i, :] = t[n:, :b]
            t2_ref[:, i + 1, :] = t[n:, b:]
    else:
        for i in range(hc):
            t = jnp.dot(S12, fs_ref[:, i, :],
                        preferred_element_type=F32).astype(BF16)
            t1_ref[:, i, :] = t[:n]
            t2_ref[:, i, :] = t[n:]


def _gate_kernel(nblk, b, h, pad, s_ref, t1_ref, t2_ref, xc_ref, wg_ref,
                 zs_ref, r_ref):
    zpad = jnp.zeros((pad, b), BF16)
    for i in range(nblk):
        s = s_ref[i]
        feat = jnp.concatenate(
            [s.astype(BF16), t1_ref[i], t2_ref[i], xc_ref[i], zpad], axis=0)
        zr = jax.nn.sigmoid(
            lax.dot_general(wg_ref[i], feat, (((0,), (0,)), ((), ())),
                            preferred_element_type=F32))
        z = zr[:h]
        r = zr[h:]
        zs_ref[i] = z * s
        r_ref[i] = r.astype(BF16)


def _cand_kernel(nblk, b, h, pad, zs_ref, u1_ref, u2_ref, xc_ref, r_ref,
                 s_ref, wu_ref, h_ref):
    zpad = jnp.zeros((pad, b), BF16)
    for i in range(nblk):
        feat = jnp.concatenate(
            [zs_ref[i].astype(BF16), u1_ref[i], u2_ref[i], xc_ref[i], zpad],
            axis=0)
        hc = jnp.tanh(
            lax.dot_general(wu_ref[i], feat, (((0,), (0,)), ((), ())),
                            preferred_element_type=F32))
        r = r_ref[i].astype(F32)
        s = s_ref[i]
        h_ref[i] = r * s + (1.0 - r) * hc


def kernel(x, state, node_embeddings, gate_w, gate_b, update_w, update_b):
    B, N, C_in = x.shape
    H = state.shape[-1]
    D = node_embeddings.shape[-1]
    K = 3
    C = C_in + H
    KC = K * C
    KCP = -(-KC // 128) * 128
    XCW = K * C_in + 2
    FPAD = KCP - K * H - XCW

    def pool(w, bias, O):
        w3 = w.reshape(D, KC, O).astype(F32)
        parts = [w3[:, k * C + C_in:(k + 1) * C, :] for k in range(K)]
        parts += [w3[:, k * C:k * C + C_in, :] for k in range(K)]
        parts.append(bias[:, None, :].astype(F32))
        parts.append(jnp.zeros((D, KCP - KC - 1, O), F32))
        return jnp.concatenate(parts, axis=1).reshape(D, KCP * O)

    pg = pool(gate_w, gate_b, 2 * H)
    pu = pool(update_w, update_b, H)
    wf = update_w.reshape(D, KC * H).astype(F32)

    sT = jnp.transpose(state, (1, 2, 0))
    xT = jnp.transpose(x, (1, 2, 0)).astype(BF16)

    par = pltpu.CompilerParams(
        dimension_semantics=("parallel", "arbitrary"),
        vmem_limit_bytes=64 * 1024 * 1024)

    s12, xc = pl.pallas_call(
        functools.partial(_supports_kernel, C_in),
        out_shape=(jax.ShapeDtypeStruct((2 * N, N), F32),
                   jax.ShapeDtypeStruct((N, XCW, B), BF16)),
        grid=(1,),
        in_specs=[pl.BlockSpec((N, D), lambda i: (0, 0)),
                  pl.BlockSpec((N, C_in, B), lambda i: (0, 0, 0))],
        out_specs=[pl.BlockSpec((2 * N, N), lambda i: (0, 0)),
                   pl.BlockSpec((N, XCW, B), lambda i: (0, 0, 0))],
        compiler_params=pltpu.CompilerParams(
            dimension_semantics=("arbitrary",)),
    )(node_embeddings, xT)

    NBW = 4 if N % 4 == 0 else 1
    nw = N // NBW
    wg2, wu2, wout2 = pl.pallas_call(
        _node_weights_kernel,
        out_shape=(jax.ShapeDtypeStruct((N, KCP * 2 * H), BF16),
                   jax.ShapeDtypeStruct((N, KCP * H), BF16),
                   jax.ShapeDtypeStruct((N, KC * H), F32)),
        grid=(2, NBW // 2),
        in_specs=[pl.BlockSpec((nw, D), lambda c, i: (c * (NBW // 2) + i, 0)),
                  pl.BlockSpec((D, KCP * 2 * H), lambda c, i: (0, 0)),
                  pl.BlockSpec((D, KCP * H), lambda c, i: (0, 0)),
                  pl.BlockSpec((D, KC * H), lambda c, i: (0, 0))],
        out_specs=[pl.BlockSpec((nw, KCP * 2 * H),
                                lambda c, i: (c * (NBW // 2) + i, 0)),
                   pl.BlockSpec((nw, KCP * H),
                                lambda c, i: (c * (NBW // 2) + i, 0)),
                   pl.BlockSpec((nw, KC * H),
                                lambda c, i: (c * (NBW // 2) + i, 0))],
        compiler_params=par,
    )(node_embeddings, pg, pu, wf)
    w_out = wout2.reshape(N, K, C, H)
    wg3 = wg2.reshape(N, KCP, 2 * H)
    wu3 = wu2.reshape(N, KCP, H)

    NCH = 8
    hc = H // NCH
    t1, t2 = pl.pallas_call(
        functools.partial(_conv_kernel, hc, N),
        out_shape=(jax.ShapeDtypeStruct((N, H, B), BF16),
                   jax.ShapeDtypeStruct((N, H, B), BF16)),
        grid=(2, NCH // 2),
        in_specs=[pl.BlockSpec((2 * N, N), lambda c, j: (0, 0)),
                  pl.BlockSpec((N, hc, B),
                               lambda c, j: (0, c * (NCH // 2) + j, 0))],
        out_specs=[pl.BlockSpec((N, hc, B),
                                lambda c, j: (0, c * (NCH // 2) + j, 0)),
                   pl.BlockSpec((N, hc, B),
                                lambda c, j: (0, c * (NCH // 2) + j, 0))],
        compiler_params=par,
    )(s12, sT)

    NBLK = 1
    for cand_blk in (16, 8, 4, 2):
        if N % cand_blk == 0 and (N // cand_blk) % 2 == 0:
            NBLK = cand_blk
            break
    G = N // NBLK
    zs, r3 = pl.pallas_call(
        functools.partial(_gate_kernel, NBLK, B, H, FPAD),
        out_shape=(jax.ShapeDtypeStruct((N, H, B), F32),
                   jax.ShapeDtypeStruct((N, H, B), BF16)),
        grid=(2, G // 2),
        in_specs=[pl.BlockSpec((NBLK, H, B),
                               lambda c, j: (c * (G // 2) + j, 0, 0)),
                  pl.BlockSpec((NBLK, H, B),
                               lambda c, j: (c * (G // 2) + j, 0, 0)),
                  pl.BlockSpec((NBLK, H, B),
                               lambda c, j: (c * (G // 2) + j, 0, 0)),
                  pl.BlockSpec((NBLK, XCW, B),
                               lambda c, j: (c * (G // 2) + j, 0, 0)),
                  pl.BlockSpec((NBLK, KCP, 2 * H),
                               lambda c, j: (c * (G // 2) + j, 0, 0))],
        out_specs=[pl.BlockSpec((NBLK, H, B),
                                lambda c, j: (c * (G // 2) + j, 0, 0)),
                   pl.BlockSpec((NBLK, H, B),
                                lambda c, j: (c * (G // 2) + j, 0, 0))],
        compiler_params=par,
    )(sT, t1, t2, xc, wg3)

    u1, u2 = pl.pallas_call(
        functools.partial(_conv_kernel, hc, N),
        out_shape=(jax.ShapeDtypeStruct((N, H, B), BF16),
                   jax.ShapeDtypeStruct((N, H, B), BF16)),
        grid=(2, NCH // 2),
        in_specs=[pl.BlockSpec((2 * N, N), lambda c, j: (0, 0)),
                  pl.BlockSpec((N, hc, B),
                               lambda c, j: (0, c * (NCH // 2) + j, 0))],
        out_specs=[pl.BlockSpec((N, hc, B),
                                lambda c, j: (0, c * (NCH // 2) + j, 0)),
                   pl.BlockSpec((N, hc, B),
                                lambda c, j: (0, c * (NCH // 2) + j, 0))],
        compiler_params=par,
    )(s12, zs)

    hb = pl.pallas_call(
        functools.partial(_cand_kernel, NBLK, B, H, FPAD),
        out_shape=jax.ShapeDtypeStruct((N, H, B), F32),
        grid=(2, G // 2),
        in_specs=[pl.BlockSpec((NBLK, H, B),
                               lambda c, j: (c * (G // 2) + j, 0, 0)),
                  pl.BlockSpec((NBLK, H, B),
                               lambda c, j: (c * (G // 2) + j, 0, 0)),
                  pl.BlockSpec((NBLK, H, B),
                               lambda c, j: (c * (G // 2) + j, 0, 0)),
                  pl.BlockSpec((NBLK, XCW, B),
                               lambda c, j: (c * (G // 2) + j, 0, 0)),
                  pl.BlockSpec((NBLK, H, B),
                               lambda c, j: (c * (G // 2) + j, 0, 0)),
                  pl.BlockSpec((NBLK, H, B),
                               lambda c, j: (c * (G // 2) + j, 0, 0)),
                  pl.BlockSpec((NBLK, KCP, H),
                               lambda c, j: (c * (G // 2) + j, 0, 0))],
        out_specs=pl.BlockSpec((NBLK, H, B),
                               lambda c, j: (c * (G // 2) + j, 0, 0)),
        compiler_params=par,
    )(zs, u1, u2, xc, r3, sT, wu3)

    h = jnp.transpose(hb, (2, 0, 1))
    return h, w_out
